# Initial kernel scaffold; baseline (speedup 1.0000x reference)
import jax
import jax.numpy as jnp
from jax.experimental import pallas as pl


def kernel(x, nf4_lut, absmax, weight_quant_packed):
    raise NotImplementedError("write your pallas kernel here")


# speedup vs baseline: 2.7253x; 2.7253x over previous
import functools

import jax
import jax.numpy as jnp
from jax import lax
from jax.experimental import pallas as pl
from jax.experimental.pallas import tpu as pltpu
from jax.experimental.pallas import tpu_sc as plsc

_INFO = plsc.get_sparse_core_info()
_NC, _NS = _INFO.num_cores, _INFO.num_subcores
_NW = _NC * _NS


@functools.lru_cache(maxsize=None)
def _make_gather_dequant(V, N, CH):
    assert N % (_NW * CH) == 0
    n_chunks = N // (_NW * CH)
    b_per_w = N // _NW
    mesh = plsc.VectorSubcoreMesh(core_axis_name="c", subcore_axis_name="s")

    @functools.partial(
        pl.kernel,
        mesh=mesh,
        compiler_params=pltpu.CompilerParams(
            needs_layout_passes=False, use_tc_tiling_on_sc=False),
        out_type=jax.ShapeDtypeStruct((N * 64,), jnp.float32),
        scratch_types=[
            pltpu.VMEM((n_chunks, CH), jnp.int32),
            pltpu.VMEM((CH, 16), jnp.int32),
            pltpu.VMEM((CH * 64,), jnp.float32),
            pltpu.VMEM((16,), jnp.float32),
            pltpu.SemaphoreType.DMA,
            pltpu.SemaphoreType.DMA,
        ],
    )
    def k(table_hbm, idx_hbm, lut_hbm, out_hbm, idx_v, rows_v, out_v, lut_v,
          gsem, osem):
        wid = lax.axis_index("s") * _NC + lax.axis_index("c")
        pltpu.sync_copy(lut_hbm, lut_v)
        pltpu.sync_copy(idx_hbm.at[wid], idx_v)

        iota = lax.iota(jnp.int32, 16)
        half = iota >> 3
        c_shift = 4 - 4 * half
        c_oidx = 8 * (iota & 7) + half
        shifts = [c_shift + 4 * kp for kp in (0, 2, 4, 6)]
        oidxs = [c_oidx + kp for kp in (0, 2, 4, 6)]

        for c in range(n_chunks):
            pltpu.async_copy(table_hbm.at[idx_v.at[c]], rows_v, gsem).wait()

            def row_body(r, carry):
                words = rows_v[r]
                base = r * 64
                for i in range(4):
                    codes = (words >> shifts[i]) & 15
                    vals = plsc.load_gather(lut_v, [codes])
                    plsc.store_scatter(out_v, [oidxs[i] + base], vals)
                return carry

            lax.fori_loop(0, CH, row_body, 0)

            out_base = pl.multiple_of((wid * b_per_w + c * CH) * 64, 4096)
            pltpu.async_copy(
                out_v, out_hbm.at[pl.ds(out_base, CH * 64)], osem).wait()

    return k


def kernel(x, nf4_lut, absmax, weight_quant_packed):
    B, L = x.shape
    V, Dh = weight_quant_packed.shape
    D = 2 * Dh
    N = B * L
    CH = 640
    words = lax.bitcast_convert_type(
        weight_quant_packed.reshape(V, Dh // 4, 4), jnp.int32)
    table = jnp.concatenate([words, words], axis=1)
    idx3 = x.reshape(_NW, N // (_NW * CH), CH)
    scaled_lut = (nf4_lut * absmax).astype(jnp.float32)
    out_flat = _make_gather_dequant(V, N, CH)(table, idx3, scaled_lut)
    return out_flat.reshape(B, L, D)
